# Initial kernel scaffold; baseline (speedup 1.0000x reference)
import jax
import jax.numpy as jnp
from jax.experimental import pallas as pl


def kernel(z, pos, emb, mlp_w1, mlp_b1, mlp_w2, mlp_b2, conv_w1, conv_w2, conv_b2, lin_w, lin_b, out_w1, out_b1, out_w2, out_b2):
    raise NotImplementedError("write your pallas kernel here")


# speedup vs baseline: 1.1884x; 1.1884x over previous
import functools

import jax
import jax.numpy as jnp
import numpy as np
from jax.experimental import pallas as pl

HIDDEN = 128
NUM_FILTERS = 128
NUM_INTERACTIONS = 6
NUM_GAUSSIANS = 50
CUTOFF = 10.0
MAX_NEIGHBORS = 16
N_NODES = 10000


def _ssp(x):
    return jax.nn.softplus(x) - jnp.log(2.0)


def _knn(pos):
    N = pos.shape[0]
    sq = jnp.sum(pos * pos, axis=-1)
    nbrs = []
    chunk = 1000
    for start in range(0, N, chunk):
        p = pos[start:start + chunk]
        c = p.shape[0]
        d2 = sq[start:start + c][:, None] + sq[None, :] - 2.0 * (p @ pos.T)
        idx = jnp.arange(start, start + c)
        d2 = d2.at[jnp.arange(c), idx].set(jnp.inf)
        _, nbr = jax.lax.top_k(-d2, MAX_NEIGHBORS)
        nbrs.append(nbr)
    return jnp.concatenate(nbrs, axis=0)


def _readout_kernel(h_ref, w1_ref, b1_ref, w2_ref, b2_ref, o_ref):
    t = _ssp(h_ref[...] @ w1_ref[...] + b1_ref[...])
    o_ref[...] = t @ w2_ref[...] + b2_ref[...]


def kernel(z, pos, emb, mlp_w1, mlp_b1, mlp_w2, mlp_b2, conv_w1, conv_w2,
           conv_b2, lin_w, lin_b, out_w1, out_b1, out_w2, out_b2):
    h = emb[z]
    N = h.shape[0]
    nbr = _knn(pos)
    d = pos[nbr] - pos[:, None, :]
    edge_weight = jnp.sqrt(jnp.sum(d * d, axis=-1))
    offset = jnp.linspace(0.0, CUTOFF, NUM_GAUSSIANS)
    coeff = -0.5 / (offset[1] - offset[0]) ** 2
    ea = jnp.exp(coeff * (edge_weight[..., None] - offset) ** 2)
    C = 0.5 * (jnp.cos(edge_weight * jnp.pi / CUTOFF) + 1.0)

    for l in range(NUM_INTERACTIONS):
        W = _ssp(ea @ mlp_w1[l] + mlp_b1[l]) @ mlp_w2[l] + mlp_b2[l]
        W = W * C[..., None]
        hw = h @ conv_w1[l]
        xj = hw[nbr]
        agg = jnp.sum(xj * W, axis=1)
        hc = _ssp(agg @ conv_w2[l] + conv_b2[l])
        hc = hc @ lin_w[l] + lin_b[l]
        h = h + hc

    out = pl.pallas_call(
        _readout_kernel,
        out_shape=jax.ShapeDtypeStruct((N, 1), jnp.float32),
    )(h, out_w1, out_b1[None, :], out_w2, out_b2[None, :])
    return out


# speedup vs baseline: 3.2541x; 2.7382x over previous
import functools

import jax
import jax.numpy as jnp
import numpy as np
from jax.experimental import pallas as pl
from jax.experimental.pallas import tpu as pltpu

HIDDEN = 128
NUM_FILTERS = 128
NUM_INTERACTIONS = 6
NUM_GAUSSIANS = 50
CUTOFF = 10.0
MAX_NEIGHBORS = 16
N_NODES = 10000


def _ssp(x):
    return jax.nn.softplus(x) - jnp.log(2.0)


_KNN_R = 400
_BIG = 3.0e9
_BIGI = 1 << 30


_KNN_CH = 1280


def _knn_body(p_ref, pt_ref, nbr_ref, d2o_ref, scr):
    i = pl.program_id(0)
    p = p_ref[...]
    R = p.shape[0]
    Np = pt_ref.shape[1]
    CH = _KNN_CH
    NCH = Np // CH
    p16 = p.astype(jnp.bfloat16)
    sq_r = jnp.sum(p * p, axis=1, keepdims=True)
    for c in range(NCH):
        pt_c = pt_ref[:, c * CH:(c + 1) * CH]
        sq_c = jnp.sum(pt_c * pt_c, axis=0, keepdims=True)
        dot = jax.lax.dot_general(
            p16, pt_c.astype(jnp.bfloat16), (((1,), (0,)), ((), ())),
            preferred_element_type=jnp.float32)
        d2c = (sq_r + sq_c) - 2.0 * dot
        colg = jax.lax.broadcasted_iota(jnp.int32, (R, CH), 1) + c * CH
        rowg = jax.lax.broadcasted_iota(jnp.int32, (R, CH), 0) + i * R
        scr[:, c * CH:(c + 1) * CH] = jnp.where(colg == rowg, _BIG, d2c)
    nbrs, vals = [], []
    for _ in range(MAX_NEIGHBORS):
        cmins = [jnp.min(scr[:, c * CH:(c + 1) * CH], axis=1, keepdims=True)
                 for c in range(NCH)]
        m = jnp.min(jnp.concatenate(cmins, axis=1), axis=1, keepdims=True)
        am = None
        for c in range(NCH):
            d2c = scr[:, c * CH:(c + 1) * CH]
            colg = jax.lax.broadcasted_iota(jnp.int32, (R, CH), 1) + c * CH
            amc = jnp.min(jnp.where(d2c == m, colg, _BIGI), axis=1,
                          keepdims=True)
            am = amc if am is None else jnp.minimum(am, amc)
        for c in range(NCH):
            d2c = scr[:, c * CH:(c + 1) * CH]
            colg = jax.lax.broadcasted_iota(jnp.int32, (R, CH), 1) + c * CH
            scr[:, c * CH:(c + 1) * CH] = jnp.where(colg == am, _BIG, d2c)
        nbrs.append(am)
        vals.append(m)
    nbr_ref[...] = jnp.concatenate(nbrs, axis=1)
    d2o_ref[...] = jnp.concatenate(vals, axis=1)


def _knn(pos):
    N = pos.shape[0]
    R = _KNN_R
    Np = ((N + _KNN_CH - 1) // _KNN_CH) * _KNN_CH
    pos_pad = jnp.pad(pos, ((0, 0), (0, 5)))
    posT_pad = jnp.pad(pos_pad.T, ((0, 0), (0, Np - N)), constant_values=100.0)
    nbr, d2 = pl.pallas_call(
        _knn_body,
        grid=(N // R,),
        in_specs=[
            pl.BlockSpec((R, 8), lambda i: (i, 0)),
            pl.BlockSpec((8, Np), lambda i: (0, 0)),
        ],
        out_specs=[
            pl.BlockSpec((R, MAX_NEIGHBORS), lambda i: (i, 0)),
            pl.BlockSpec((R, MAX_NEIGHBORS), lambda i: (i, 0)),
        ],
        out_shape=[
            jax.ShapeDtypeStruct((N, MAX_NEIGHBORS), jnp.int32),
            jax.ShapeDtypeStruct((N, MAX_NEIGHBORS), jnp.float32),
        ],
        scratch_shapes=[pltpu.VMEM((R, Np), jnp.float32)],
    )(pos_pad, posT_pad)
    return nbr, d2


def _readout_kernel(h_ref, w1_ref, b1_ref, w2_ref, b2_ref, o_ref):
    t = _ssp(h_ref[...] @ w1_ref[...] + b1_ref[...])
    o_ref[...] = t @ w2_ref[...] + b2_ref[...]


def kernel(z, pos, emb, mlp_w1, mlp_b1, mlp_w2, mlp_b2, conv_w1, conv_w2,
           conv_b2, lin_w, lin_b, out_w1, out_b1, out_w2, out_b2):
    h = emb[z]
    N = h.shape[0]
    nbr, d2 = _knn(pos)
    dd = pos[nbr] - pos[:, None, :]
    edge_weight = jnp.sqrt(jnp.sum(dd * dd, axis=-1))
    offset = jnp.linspace(0.0, CUTOFF, NUM_GAUSSIANS)
    coeff = -0.5 / (offset[1] - offset[0]) ** 2
    ea = jnp.exp(coeff * (edge_weight[..., None] - offset) ** 2)
    C = 0.5 * (jnp.cos(edge_weight * jnp.pi / CUTOFF) + 1.0)

    for l in range(NUM_INTERACTIONS):
        W = _ssp(ea @ mlp_w1[l] + mlp_b1[l]) @ mlp_w2[l] + mlp_b2[l]
        W = W * C[..., None]
        hw = h @ conv_w1[l]
        xj = hw[nbr]
        agg = jnp.sum(xj * W, axis=1)
        hc = _ssp(agg @ conv_w2[l] + conv_b2[l])
        hc = hc @ lin_w[l] + lin_b[l]
        h = h + hc

    out = pl.pallas_call(
        _readout_kernel,
        out_shape=jax.ShapeDtypeStruct((N, 1), jnp.float32),
    )(h, out_w1, out_b1[None, :], out_w2, out_b2[None, :])
    return out
